# baseline (device time: 27215 ns/iter reference)
import jax
import jax.numpy as jnp
from jax import lax
from jax.experimental import pallas as pl
from jax.experimental.pallas import tpu as pltpu

N_DEV = 4
SQ = 256
D_MODEL = 1024
HALF = D_MODEL // 2
HEADS_PER_SHARD = 8
DH = 128
SCALE = 0.08838834764831843


def kernel(x, Wq, Wo, Wk, Wv):
    def body(x_ref, wq_ref, wo_ref, wk_ref, wv_ref, out_ref,
             wo_vmem, send_ref, recv_ref, load_sem, send_sems, recv_sems):
        my_pos = lax.axis_index("i")
        pa = my_pos ^ 1
        pb = my_pos ^ 3

        wo_cp = pltpu.make_async_copy(wo_ref, wo_vmem, load_sem)
        wo_cp.start()

        barrier_sem = pltpu.get_barrier_semaphore()
        for nbr in [pa, pb]:
            pl.semaphore_signal(
                barrier_sem, inc=1,
                device_id=(nbr,), device_id_type=pl.DeviceIdType.MESH,
            )
        pl.semaphore_wait(barrier_sem, 2)

        bf16 = jnp.bfloat16
        xs = x_ref[0].astype(bf16)
        q = jnp.dot(xs, wq_ref[...].astype(bf16),
                    preferred_element_type=jnp.float32)
        k = jnp.dot(xs, wk_ref[...].astype(bf16),
                    preferred_element_type=jnp.float32)
        v = jnp.dot(xs, wv_ref[...].astype(bf16),
                    preferred_element_type=jnp.float32)

        outs = []
        for h in range(HEADS_PER_SHARD):
            sl = slice(h * DH, (h + 1) * DH)
            s = jnp.dot(q[:, sl].astype(bf16), k[:, sl].astype(bf16).T,
                        preferred_element_type=jnp.float32) * SCALE
            m = jnp.max(s, axis=-1, keepdims=True)
            p = jnp.exp(s - m)
            l = jnp.sum(p, axis=-1, keepdims=True)
            outs.append(jnp.dot(p.astype(bf16), v[:, sl].astype(bf16),
                                preferred_element_type=jnp.float32) / l)
        attn = jnp.concatenate(outs, axis=1).astype(bf16)

        def xchg(step, half, target, src_sem_i):
            return pltpu.make_async_remote_copy(
                src_ref=send_ref.at[step, half],
                dst_ref=recv_ref.at[step, half],
                send_sem=send_sems.at[src_sem_i],
                recv_sem=recv_sems.at[src_sem_i],
                device_id=(target,),
                device_id_type=pl.DeviceIdType.MESH,
            )

        wo_cp.wait()
        send_ref[0, 0] = jnp.dot(attn, wo_vmem[:, :HALF].astype(bf16),
                                 preferred_element_type=jnp.float32).astype(bf16)
        r1a = xchg(0, 0, pa, 0)
        r1a.start()
        send_ref[0, 1] = jnp.dot(attn, wo_vmem[:, HALF:].astype(bf16),
                                 preferred_element_type=jnp.float32).astype(bf16)
        r1b = xchg(0, 1, pb, 1)
        r1b.start()

        f32 = jnp.float32
        r1a.wait()
        red0 = send_ref[0, 0].astype(f32) + recv_ref[0, 0].astype(f32)
        send_ref[1, 0] = red0.astype(bf16)
        r2a = xchg(1, 0, pb, 2)
        r2a.start()
        r1b.wait()
        red1 = send_ref[0, 1].astype(f32) + recv_ref[0, 1].astype(f32)
        send_ref[1, 1] = red1.astype(bf16)
        r2b = xchg(1, 1, pa, 3)
        r2b.start()

        r2a.wait()
        out_ref[0, :, :HALF] = red0 + recv_ref[1, 0].astype(f32)
        r2b.wait()
        out_ref[0, :, HALF:] = red1 + recv_ref[1, 1].astype(f32)

    return pl.pallas_call(
        body,
        out_shape=jax.ShapeDtypeStruct((1, SQ, D_MODEL), jnp.float32),
        in_specs=[
            pl.BlockSpec(memory_space=pltpu.VMEM),
            pl.BlockSpec(memory_space=pltpu.VMEM),
            pl.BlockSpec(memory_space=pltpu.HBM),
            pl.BlockSpec(memory_space=pltpu.VMEM),
            pl.BlockSpec(memory_space=pltpu.VMEM),
        ],
        out_specs=pl.BlockSpec(memory_space=pltpu.VMEM),
        scratch_shapes=[
            pltpu.VMEM((D_MODEL, D_MODEL), jnp.float32),
            pltpu.VMEM((2, 2, SQ, HALF), jnp.bfloat16),
            pltpu.VMEM((2, 2, SQ, HALF), jnp.bfloat16),
            pltpu.SemaphoreType.DMA,
            pltpu.SemaphoreType.DMA((4,)),
            pltpu.SemaphoreType.DMA((4,)),
        ],
        compiler_params=pltpu.CompilerParams(collective_id=0),
    )(x, Wq, Wo, Wk, Wv)


# device time: 25196 ns/iter; 1.0801x vs baseline; 1.0801x over previous
import jax
import jax.numpy as jnp
from jax import lax
from jax.experimental import pallas as pl
from jax.experimental.pallas import tpu as pltpu

N_DEV = 4
SQ = 256
D_MODEL = 1024
HALF = D_MODEL // 2
HEADS_PER_SHARD = 8
DH = 128
NBLK = 2
RB = SQ // NBLK
SCALE = 0.08838834764831843


def kernel(x, Wq, Wo, Wk, Wv):
    def body(x_ref, wq_ref, wo_ref, wk_ref, wv_ref, out_ref,
             send_ref, recv_ref, send_sems, recv_sems):
        my_pos = lax.axis_index("i")
        pa = my_pos ^ 1
        pb = my_pos ^ 3

        barrier_sem = pltpu.get_barrier_semaphore()
        for nbr in [pa, pb]:
            pl.semaphore_signal(
                barrier_sem, inc=1,
                device_id=(nbr,), device_id_type=pl.DeviceIdType.MESH,
            )
        pl.semaphore_wait(barrier_sem, 2)

        bf16 = jnp.bfloat16
        f32 = jnp.float32
        xs = x_ref[0].astype(bf16)
        q = jnp.dot(xs, wq_ref[...].astype(bf16), preferred_element_type=f32)
        k = jnp.dot(xs, wk_ref[...].astype(bf16), preferred_element_type=f32)
        v = jnp.dot(xs, wv_ref[...].astype(bf16), preferred_element_type=f32)
        kb = k.astype(bf16)
        vb = v.astype(bf16)
        wo0 = wo_ref[:, :HALF].astype(bf16)
        wo1 = wo_ref[:, HALF:].astype(bf16)

        def xchg(step, half, blk, target):
            i = step * 4 + half * 2 + blk
            return pltpu.make_async_remote_copy(
                src_ref=send_ref.at[step, half, blk],
                dst_ref=recv_ref.at[step, half, blk],
                send_sem=send_sems.at[i],
                recv_sem=recv_sems.at[i],
                device_id=(target,),
                device_id_type=pl.DeviceIdType.MESH,
            )

        step1 = []
        for b in range(NBLK):
            rows = slice(b * RB, (b + 1) * RB)
            outs = []
            for h in range(HEADS_PER_SHARD):
                sl = slice(h * DH, (h + 1) * DH)
                s = jnp.dot(q[rows, sl].astype(bf16), kb[:, sl].T,
                            preferred_element_type=f32) * SCALE
                m = jnp.max(s, axis=-1, keepdims=True)
                p = jnp.exp(s - m)
                l = jnp.sum(p, axis=-1, keepdims=True)
                outs.append(jnp.dot(p.astype(bf16), vb[:, sl],
                                    preferred_element_type=f32) / l)
            attn = jnp.concatenate(outs, axis=1).astype(bf16)
            send_ref[0, 0, b] = jnp.dot(
                attn, wo0, preferred_element_type=f32).astype(bf16)
            ra = xchg(0, 0, b, pa)
            ra.start()
            send_ref[0, 1, b] = jnp.dot(
                attn, wo1, preferred_element_type=f32).astype(bf16)
            rb_ = xchg(0, 1, b, pb)
            rb_.start()
            step1.append((ra, rb_))

        step2 = []
        reds = []
        for b in range(NBLK):
            ra, rb_ = step1[b]
            ra.wait()
            red0 = (send_ref[0, 0, b].astype(f32)
                    + recv_ref[0, 0, b].astype(f32))
            send_ref[1, 0, b] = red0.astype(bf16)
            r2a = xchg(1, 0, b, pb)
            r2a.start()
            rb_.wait()
            red1 = (send_ref[0, 1, b].astype(f32)
                    + recv_ref[0, 1, b].astype(f32))
            send_ref[1, 1, b] = red1.astype(bf16)
            r2b = xchg(1, 1, b, pa)
            r2b.start()
            step2.append((r2a, r2b))
            reds.append((red0, red1))

        for b in range(NBLK):
            rows = slice(b * RB, (b + 1) * RB)
            r2a, r2b = step2[b]
            red0, red1 = reds[b]
            r2a.wait()
            out_ref[0, rows, :HALF] = red0 + recv_ref[1, 0, b].astype(f32)
            r2b.wait()
            out_ref[0, rows, HALF:] = red1 + recv_ref[1, 1, b].astype(f32)

    return pl.pallas_call(
        body,
        out_shape=jax.ShapeDtypeStruct((1, SQ, D_MODEL), jnp.float32),
        in_specs=[
            pl.BlockSpec(memory_space=pltpu.VMEM),
            pl.BlockSpec(memory_space=pltpu.VMEM),
            pl.BlockSpec(memory_space=pltpu.VMEM),
            pl.BlockSpec(memory_space=pltpu.VMEM),
            pl.BlockSpec(memory_space=pltpu.VMEM),
        ],
        out_specs=pl.BlockSpec(memory_space=pltpu.VMEM),
        scratch_shapes=[
            pltpu.VMEM((2, 2, NBLK, RB, HALF), jnp.bfloat16),
            pltpu.VMEM((2, 2, NBLK, RB, HALF), jnp.bfloat16),
            pltpu.SemaphoreType.DMA((8,)),
            pltpu.SemaphoreType.DMA((8,)),
        ],
        compiler_params=pltpu.CompilerParams(collective_id=0),
    )(x, Wq, Wo, Wk, Wv)
